# baseline (speedup 1.0000x reference)
import functools

import jax
import jax.numpy as jnp
from jax import lax
from jax.experimental import pallas as pl
from jax.experimental.pallas import tpu as pltpu
from jax.experimental.pallas import tpu_sc as plsc

B, Q, M = 32, 1024, 512
W_EXIST, W_RECON, W_DIAG = 1.0, 5.0, 1.0
L = 16
NCORES, NSUB = 2, 16


def _prep_body(pe_ref, s_ref, g_ref, sums_ref):
    pe = pe_ref[...]
    s_ref[...] = 1.0 / (1.0 + jnp.exp(-pe))
    mx = jnp.maximum(pe, 0.0)
    lg = jnp.log1p(jnp.exp(-jnp.abs(pe)))
    bce0 = mx + lg
    bce1 = (mx - pe) + lg
    pos = jnp.float32(float(M))
    w_neg = pos / ((jnp.float32(float(Q)) - pos) + jnp.float32(1e-6))
    g_ref[...] = bce1 - w_neg * bce0
    sums_ref[0] = w_neg * jnp.sum(bce0)


_prep = pl.pallas_call(
    _prep_body,
    out_shape=[
        jax.ShapeDtypeStruct((B, Q), jnp.float32),
        jax.ShapeDtypeStruct((B, Q), jnp.float32),
        jax.ShapeDtypeStruct((1,), jnp.float32),
    ],
    out_specs=[
        pl.BlockSpec(memory_space=pltpu.VMEM),
        pl.BlockSpec(memory_space=pltpu.VMEM),
        pl.BlockSpec(memory_space=pltpu.SMEM),
    ],
)


def _sc_body(pp_hbm, s_hbm, g_hbm, tp_hbm, out_hbm,
             xy_v, x_v, y_v, m_v, g_v, uv_v, u_v, v_v, src_v, o_v):
    b = lax.axis_index("c") * NSUB + lax.axis_index("s")
    pltpu.sync_copy(pp_hbm.at[b], xy_v)
    pltpu.sync_copy(s_hbm.at[b], m_v)
    pltpu.sync_copy(g_hbm.at[b], g_v)
    pltpu.sync_copy(tp_hbm.at[b], uv_v)

    iota = lax.iota(jnp.int32, L)
    lane0 = iota == 0
    neg_inf = jnp.float32(-jnp.inf)
    perms = [iota ^ (1 << r) for r in range(4)]
    iota2 = iota * 2

    def deint_pp(c, dacc):
        sl = pl.ds(c * L, L)
        idx = iota2 + c * (2 * L)
        xs = plsc.load_gather(xy_v, [idx])
        ys = plsc.load_gather(xy_v, [idx + 1])
        x_v[sl] = xs
        y_v[sl] = ys
        diff = ys - xs
        return dacc + diff * diff

    dd = lax.fori_loop(0, Q // L, deint_pp,
                       jnp.zeros((L,), jnp.float32), unroll=4)

    def deint_tp(c, carry):
        sl = pl.ds(c * L, L)
        idx = iota2 + c * (2 * L)
        u_v[sl] = plsc.load_gather(uv_v, [idx])
        v_v[sl] = plsc.load_gather(uv_v, [idx + 1])
        return carry

    lax.fori_loop(0, M // L, deint_tp, jnp.int32(0), unroll=4)

    def argmin16(best, bidx):
        for p in perms:
            v2 = best.at[p].get(mode="promise_in_bounds")
            i2 = bidx.at[p].get(mode="promise_in_bounds")
            lt = (v2 < best) | ((v2 == best) & (i2 < bidx))
            best = jnp.where(lt, v2, best)
            bidx = jnp.where(lt, i2, bidx)
        return best, bidx

    def outer(jc, carry):
        uc = u_v[pl.ds(jc * L, L)]
        vc = v_v[pl.ds(jc * L, L)]
        for t in range(L):
            j = jc * L + t
            ub = jnp.full((L,), uc[t], jnp.float32)
            vb = jnp.full((L,), vc[t], jnp.float32)

            def chunk(c, bc):
                best, bch = bc
                sl = pl.ds(c * L, L)
                dx = x_v[sl] - ub
                dy = y_v[sl] - vb
                key = (dx * dx + dy * dy) - m_v[sl]
                lt = key < best
                cc = jnp.full((L,), c, jnp.int32)
                return jnp.where(lt, key, best), jnp.where(lt, cc, bch)

            best0 = jnp.full((L,), jnp.inf, jnp.float32)
            bidx0 = jnp.zeros((L,), jnp.int32)
            best, bch = lax.fori_loop(0, Q // L, chunk, (best0, bidx0),
                                      unroll=8)
            bidx = bch * L + iota
            _, miv = argmin16(best, bidx)
            plsc.store_scatter(m_v, [miv],
                               jnp.full((L,), neg_inf, jnp.float32),
                               mask=lane0)
            plsc.store_scatter(src_v, [jnp.full((L,), j, jnp.int32)], miv,
                               mask=lane0)
        return carry

    lax.fori_loop(0, M // L, outer, jnp.int32(0))

    def acc(c, carry):
        R, G, D = carry
        sl = pl.ds(c * L, L)
        idx = src_v[sl]
        gx = plsc.load_gather(x_v, [idx])
        gy = plsc.load_gather(y_v, [idx])
        gg = plsc.load_gather(g_v, [idx])
        dx = gx - u_v[sl]
        dy = gy - v_v[sl]
        gd = gy - gx
        return R + (dx * dx + dy * dy), G + gg, D + gd * gd

    z = jnp.zeros((L,), jnp.float32)
    R, G, D = lax.fori_loop(0, M // L, acc, (z, z, z), unroll=4)
    sR = jnp.sum(R)
    sG = jnp.sum(G)
    sD = jnp.sum(D)
    sDd = jnp.sum(dd)
    vec = jnp.where(iota == 0, sR,
                    jnp.where(iota == 1, sG,
                              jnp.where(iota == 2, sD,
                                        jnp.where(iota == 3, sDd,
                                                  jnp.float32(0.0)))))
    o_v[...] = vec
    pltpu.sync_copy(o_v, out_hbm.at[b])


_sc_match = functools.partial(
    pl.kernel,
    out_type=jax.ShapeDtypeStruct((B, L), jnp.float32),
    mesh=plsc.VectorSubcoreMesh(core_axis_name="c", subcore_axis_name="s"),
    compiler_params=pltpu.CompilerParams(needs_layout_passes=False),
    scratch_types=[
        pltpu.VMEM((2 * Q,), jnp.float32),
        pltpu.VMEM((Q,), jnp.float32),
        pltpu.VMEM((Q,), jnp.float32),
        pltpu.VMEM((Q,), jnp.float32),
        pltpu.VMEM((Q,), jnp.float32),
        pltpu.VMEM((2 * M,), jnp.float32),
        pltpu.VMEM((M,), jnp.float32),
        pltpu.VMEM((M,), jnp.float32),
        pltpu.VMEM((M,), jnp.int32),
        pltpu.VMEM((L,), jnp.float32),
    ],
)(_sc_body)


def kernel(pred_pairs, pred_exist, target_pairs):
    pp2 = pred_pairs.reshape(B, 2 * Q)
    tp2 = target_pairs.reshape(B, 2 * M)
    s, g, sums = _prep(pred_exist)
    part = _sc_match(pp2, s, g, tp2)
    num = jnp.float32(float(B * M))
    l_exist = (sums[0] + part[:, 1].sum()) / num
    l_recon = part[:, 0].sum() / num
    l_diag = (part[:, 3].sum() - part[:, 2].sum()) / num
    return jnp.stack([l_exist * W_EXIST, l_recon * W_RECON, l_diag * W_DIAG])


# speedup vs baseline: 1.2451x; 1.2451x over previous
import functools

import jax
import jax.numpy as jnp
from jax import lax
from jax.experimental import pallas as pl
from jax.experimental.pallas import tpu as pltpu
from jax.experimental.pallas import tpu_sc as plsc

B, Q, M = 32, 1024, 512
W_EXIST, W_RECON, W_DIAG = 1.0, 5.0, 1.0
L = 16
NCORES, NSUB = 2, 16


def _prep_body(pe_ref, s_ref, g_ref, sums_ref):
    pe = pe_ref[...]
    s_ref[...] = 1.0 / (1.0 + jnp.exp(-pe))
    mx = jnp.maximum(pe, 0.0)
    lg = jnp.log1p(jnp.exp(-jnp.abs(pe)))
    bce0 = mx + lg
    bce1 = (mx - pe) + lg
    pos = jnp.float32(float(M))
    w_neg = pos / ((jnp.float32(float(Q)) - pos) + jnp.float32(1e-6))
    g_ref[...] = bce1 - w_neg * bce0
    sums_ref[0] = w_neg * jnp.sum(bce0)


_prep = pl.pallas_call(
    _prep_body,
    out_shape=[
        jax.ShapeDtypeStruct((B, Q), jnp.float32),
        jax.ShapeDtypeStruct((B, Q), jnp.float32),
        jax.ShapeDtypeStruct((1,), jnp.float32),
    ],
    out_specs=[
        pl.BlockSpec(memory_space=pltpu.VMEM),
        pl.BlockSpec(memory_space=pltpu.VMEM),
        pl.BlockSpec(memory_space=pltpu.SMEM),
    ],
)


def _sc_body(pp_hbm, s_hbm, g_hbm, tp_hbm, out_hbm,
             xy_v, x_v, y_v, m_v, g_v, uv_v, u_v, v_v, src_v, o_v):
    b = lax.axis_index("c") * NSUB + lax.axis_index("s")
    pltpu.sync_copy(pp_hbm.at[b], xy_v)
    pltpu.sync_copy(s_hbm.at[b], m_v)
    pltpu.sync_copy(g_hbm.at[b], g_v)
    pltpu.sync_copy(tp_hbm.at[b], uv_v)

    iota = lax.iota(jnp.int32, L)
    lane0 = iota == 0
    neg_inf = jnp.float32(-jnp.inf)
    perms = [iota ^ (1 << r) for r in range(4)]
    iota2 = iota * 2

    def deint_pp(c, dacc):
        sl = pl.ds(c * L, L)
        idx = iota2 + c * (2 * L)
        xs = plsc.load_gather(xy_v, [idx])
        ys = plsc.load_gather(xy_v, [idx + 1])
        x_v[sl] = xs
        y_v[sl] = ys
        diff = ys - xs
        return dacc + diff * diff

    dd = lax.fori_loop(0, Q // L, deint_pp,
                       jnp.zeros((L,), jnp.float32), unroll=4)

    def deint_tp(c, carry):
        sl = pl.ds(c * L, L)
        idx = iota2 + c * (2 * L)
        u_v[sl] = plsc.load_gather(uv_v, [idx])
        v_v[sl] = plsc.load_gather(uv_v, [idx + 1])
        return carry

    lax.fori_loop(0, M // L, deint_tp, jnp.int32(0), unroll=4)

    def argmin16(best, bidx):
        for p in perms:
            v2 = best.at[p].get(mode="promise_in_bounds")
            i2 = bidx.at[p].get(mode="promise_in_bounds")
            lt = (v2 < best) | ((v2 == best) & (i2 < bidx))
            best = jnp.where(lt, v2, best)
            bidx = jnp.where(lt, i2, bidx)
        return best, bidx

    def outer(jc, carry):
        uc = u_v[pl.ds(jc * L, L)]
        vc = v_v[pl.ds(jc * L, L)]
        for t in range(L):
            j = jc * L + t
            ub = jnp.full((L,), uc[t], jnp.float32)
            vb = jnp.full((L,), vc[t], jnp.float32)

            def chunk(c, bc):
                best, bidx = bc
                sl = pl.ds(c * L, L)
                dx = x_v[sl] - ub
                dy = y_v[sl] - vb
                key = (dx * dx + dy * dy) - m_v[sl]
                lt = key < best
                ii = iota + c * L
                return jnp.where(lt, key, best), jnp.where(lt, ii, bidx)

            best0 = jnp.full((L,), jnp.inf, jnp.float32)
            bidx0 = jnp.zeros((L,), jnp.int32)
            best, bidx = lax.fori_loop(0, Q // L, chunk, (best0, bidx0),
                                       unroll=4)
            _, miv = argmin16(best, bidx)
            plsc.store_scatter(m_v, [miv],
                               jnp.full((L,), neg_inf, jnp.float32),
                               mask=lane0)
            plsc.store_scatter(src_v, [jnp.full((L,), j, jnp.int32)], miv,
                               mask=lane0)
        return carry

    lax.fori_loop(0, M // L, outer, jnp.int32(0))

    def acc(c, carry):
        R, G, D = carry
        sl = pl.ds(c * L, L)
        idx = src_v[sl]
        gx = plsc.load_gather(x_v, [idx])
        gy = plsc.load_gather(y_v, [idx])
        gg = plsc.load_gather(g_v, [idx])
        dx = gx - u_v[sl]
        dy = gy - v_v[sl]
        gd = gy - gx
        return R + (dx * dx + dy * dy), G + gg, D + gd * gd

    z = jnp.zeros((L,), jnp.float32)
    R, G, D = lax.fori_loop(0, M // L, acc, (z, z, z), unroll=4)
    sR = jnp.sum(R)
    sG = jnp.sum(G)
    sD = jnp.sum(D)
    sDd = jnp.sum(dd)
    vec = jnp.where(iota == 0, sR,
                    jnp.where(iota == 1, sG,
                              jnp.where(iota == 2, sD,
                                        jnp.where(iota == 3, sDd,
                                                  jnp.float32(0.0)))))
    o_v[...] = vec
    pltpu.sync_copy(o_v, out_hbm.at[b])


_sc_match = functools.partial(
    pl.kernel,
    out_type=jax.ShapeDtypeStruct((B, L), jnp.float32),
    mesh=plsc.VectorSubcoreMesh(core_axis_name="c", subcore_axis_name="s"),
    compiler_params=pltpu.CompilerParams(needs_layout_passes=False),
    scratch_types=[
        pltpu.VMEM((2 * Q,), jnp.float32),
        pltpu.VMEM((Q,), jnp.float32),
        pltpu.VMEM((Q,), jnp.float32),
        pltpu.VMEM((Q,), jnp.float32),
        pltpu.VMEM((Q,), jnp.float32),
        pltpu.VMEM((2 * M,), jnp.float32),
        pltpu.VMEM((M,), jnp.float32),
        pltpu.VMEM((M,), jnp.float32),
        pltpu.VMEM((M,), jnp.int32),
        pltpu.VMEM((L,), jnp.float32),
    ],
)(_sc_body)


def kernel(pred_pairs, pred_exist, target_pairs):
    pp2 = pred_pairs.reshape(B, 2 * Q)
    tp2 = target_pairs.reshape(B, 2 * M)
    s, g, sums = _prep(pred_exist)
    part = _sc_match(pp2, s, g, tp2)
    num = jnp.float32(float(B * M))
    l_exist = (sums[0] + part[:, 1].sum()) / num
    l_recon = part[:, 0].sum() / num
    l_diag = (part[:, 3].sum() - part[:, 2].sum()) / num
    return jnp.stack([l_exist * W_EXIST, l_recon * W_RECON, l_diag * W_DIAG])


# speedup vs baseline: 1.2545x; 1.0076x over previous
import functools

import jax
import jax.numpy as jnp
from jax import lax
from jax.experimental import pallas as pl
from jax.experimental.pallas import tpu as pltpu
from jax.experimental.pallas import tpu_sc as plsc

B, Q, M = 32, 1024, 512
W_EXIST, W_RECON, W_DIAG = 1.0, 5.0, 1.0
L = 16
NCORES, NSUB = 2, 16


def _prep_body(pe_ref, s_ref, g_ref, sums_ref):
    pe = pe_ref[...]
    s_ref[...] = 1.0 / (1.0 + jnp.exp(-pe))
    mx = jnp.maximum(pe, 0.0)
    lg = jnp.log1p(jnp.exp(-jnp.abs(pe)))
    bce0 = mx + lg
    bce1 = (mx - pe) + lg
    pos = jnp.float32(float(M))
    w_neg = pos / ((jnp.float32(float(Q)) - pos) + jnp.float32(1e-6))
    g_ref[...] = bce1 - w_neg * bce0
    sums_ref[0] = w_neg * jnp.sum(bce0)


_prep = pl.pallas_call(
    _prep_body,
    out_shape=[
        jax.ShapeDtypeStruct((B, Q), jnp.float32),
        jax.ShapeDtypeStruct((B, Q), jnp.float32),
        jax.ShapeDtypeStruct((1,), jnp.float32),
    ],
    out_specs=[
        pl.BlockSpec(memory_space=pltpu.VMEM),
        pl.BlockSpec(memory_space=pltpu.VMEM),
        pl.BlockSpec(memory_space=pltpu.SMEM),
    ],
)


def _sc_body(pp_hbm, s_hbm, g_hbm, tp_hbm, out_hbm,
             xy_v, x_v, y_v, m_v, g_v, uv_v, u_v, v_v, src_v, o_v):
    b = lax.axis_index("c") * NSUB + lax.axis_index("s")
    pltpu.sync_copy(pp_hbm.at[b], xy_v)
    pltpu.sync_copy(s_hbm.at[b], m_v)
    pltpu.sync_copy(g_hbm.at[b], g_v)
    pltpu.sync_copy(tp_hbm.at[b], uv_v)

    iota = lax.iota(jnp.int32, L)
    lane0 = iota == 0
    neg_inf = jnp.float32(-jnp.inf)
    perms = [iota ^ (1 << r) for r in range(4)]
    iota2 = iota * 2

    def deint_pp(c, dacc):
        sl = pl.ds(c * L, L)
        idx = iota2 + c * (2 * L)
        xs = plsc.load_gather(xy_v, [idx])
        ys = plsc.load_gather(xy_v, [idx + 1])
        x_v[sl] = xs
        y_v[sl] = ys
        diff = ys - xs
        return dacc + diff * diff

    dd = lax.fori_loop(0, Q // L, deint_pp,
                       jnp.zeros((L,), jnp.float32), unroll=4)

    def deint_tp(c, carry):
        sl = pl.ds(c * L, L)
        idx = iota2 + c * (2 * L)
        u_v[sl] = plsc.load_gather(uv_v, [idx])
        v_v[sl] = plsc.load_gather(uv_v, [idx + 1])
        return carry

    lax.fori_loop(0, M // L, deint_tp, jnp.int32(0), unroll=4)

    def argmin16(best, bidx):
        for p in perms:
            v2 = best.at[p].get(mode="promise_in_bounds")
            i2 = bidx.at[p].get(mode="promise_in_bounds")
            lt = (v2 < best) | ((v2 == best) & (i2 < bidx))
            best = jnp.where(lt, v2, best)
            bidx = jnp.where(lt, i2, bidx)
        return best, bidx

    def outer(jc, carry):
        uc = u_v[pl.ds(jc * L, L)]
        vc = v_v[pl.ds(jc * L, L)]
        for t in range(L):
            j = jc * L + t
            ub = jnp.full((L,), uc[t], jnp.float32)
            vb = jnp.full((L,), vc[t], jnp.float32)

            def chunk(c, bc):
                best, bidx = bc
                sl = pl.ds(c * L, L)
                dx = x_v[sl] - ub
                dy = y_v[sl] - vb
                key = (dx * dx + dy * dy) - m_v[sl]
                lt = key < best
                ii = iota + c * L
                return jnp.where(lt, key, best), jnp.where(lt, ii, bidx)

            best0 = jnp.full((L,), jnp.inf, jnp.float32)
            bidx0 = jnp.zeros((L,), jnp.int32)
            best, bidx = lax.fori_loop(0, Q // L, chunk, (best0, bidx0),
                                       unroll=8)
            _, miv = argmin16(best, bidx)
            plsc.store_scatter(m_v, [miv],
                               jnp.full((L,), neg_inf, jnp.float32),
                               mask=lane0)
            plsc.store_scatter(src_v, [jnp.full((L,), j, jnp.int32)], miv,
                               mask=lane0)
        return carry

    lax.fori_loop(0, M // L, outer, jnp.int32(0))

    def acc(c, carry):
        R, G, D = carry
        sl = pl.ds(c * L, L)
        idx = src_v[sl]
        gx = plsc.load_gather(x_v, [idx])
        gy = plsc.load_gather(y_v, [idx])
        gg = plsc.load_gather(g_v, [idx])
        dx = gx - u_v[sl]
        dy = gy - v_v[sl]
        gd = gy - gx
        return R + (dx * dx + dy * dy), G + gg, D + gd * gd

    z = jnp.zeros((L,), jnp.float32)
    R, G, D = lax.fori_loop(0, M // L, acc, (z, z, z), unroll=4)
    sR = jnp.sum(R)
    sG = jnp.sum(G)
    sD = jnp.sum(D)
    sDd = jnp.sum(dd)
    vec = jnp.where(iota == 0, sR,
                    jnp.where(iota == 1, sG,
                              jnp.where(iota == 2, sD,
                                        jnp.where(iota == 3, sDd,
                                                  jnp.float32(0.0)))))
    o_v[...] = vec
    pltpu.sync_copy(o_v, out_hbm.at[b])


_sc_match = functools.partial(
    pl.kernel,
    out_type=jax.ShapeDtypeStruct((B, L), jnp.float32),
    mesh=plsc.VectorSubcoreMesh(core_axis_name="c", subcore_axis_name="s"),
    compiler_params=pltpu.CompilerParams(needs_layout_passes=False),
    scratch_types=[
        pltpu.VMEM((2 * Q,), jnp.float32),
        pltpu.VMEM((Q,), jnp.float32),
        pltpu.VMEM((Q,), jnp.float32),
        pltpu.VMEM((Q,), jnp.float32),
        pltpu.VMEM((Q,), jnp.float32),
        pltpu.VMEM((2 * M,), jnp.float32),
        pltpu.VMEM((M,), jnp.float32),
        pltpu.VMEM((M,), jnp.float32),
        pltpu.VMEM((M,), jnp.int32),
        pltpu.VMEM((L,), jnp.float32),
    ],
)(_sc_body)


def kernel(pred_pairs, pred_exist, target_pairs):
    pp2 = pred_pairs.reshape(B, 2 * Q)
    tp2 = target_pairs.reshape(B, 2 * M)
    s, g, sums = _prep(pred_exist)
    part = _sc_match(pp2, s, g, tp2)
    num = jnp.float32(float(B * M))
    l_exist = (sums[0] + part[:, 1].sum()) / num
    l_recon = part[:, 0].sum() / num
    l_diag = (part[:, 3].sum() - part[:, 2].sum()) / num
    return jnp.stack([l_exist * W_EXIST, l_recon * W_RECON, l_diag * W_DIAG])
